# baseline (device time: 35092 ns/iter reference)
import jax
import jax.numpy as jnp
from jax import lax
from jax.experimental import pallas as pl
from jax.experimental.pallas import tpu as pltpu

N_DEV = 4
B = 2
S_LOC = 128
S = S_LOC * N_DEV
D = 512
H_LOC = 8
DH = 64

BF = jnp.bfloat16
F32 = jnp.float32


def kernel(x, Wq, Wo, Wk, Wv):
    def body(x_ref, wq_ref, wo_ref, wk_ref, wv_ref, out_ref,
             xbf_buf, xg, qf, kf, vf, osc, ysend, rsbuf,
             ag_send, ag_recv, rs_send, rs_recv):
        my = lax.axis_index("i")

        barrier_sem = pltpu.get_barrier_semaphore()
        for k in (1, 2, 3):
            pl.semaphore_signal(
                barrier_sem, inc=1,
                device_id=(lax.rem(my + k, N_DEV),),
                device_id_type=pl.DeviceIdType.MESH,
            )
        pl.semaphore_wait(barrier_sem, 3)

        xbf_buf[...] = x_ref[...].astype(BF)
        ag_rdmas = []
        for k in (1, 2, 3):
            r = 3 - k
            rdma = pltpu.make_async_remote_copy(
                src_ref=xbf_buf,
                dst_ref=xg.at[r],
                send_sem=ag_send.at[k - 1],
                recv_sem=ag_recv.at[r],
                device_id=(lax.rem(my + k, N_DEV),),
                device_id_type=pl.DeviceIdType.MESH,
            )
            rdma.start()
            ag_rdmas.append(rdma)

        wq = wq_ref[...].astype(BF)
        wk = wk_ref[...].astype(BF)
        wv = wv_ref[...].astype(BF)
        wo = wo_ref[...].astype(BF)

        def qkv_chunk(x_c, row0):
            for b in range(B):
                xcb = x_c[b]
                qf[b, pl.ds(row0, S_LOC), :] = jnp.dot(
                    xcb, wq, preferred_element_type=F32).astype(BF)
                kf[b, pl.ds(row0, S_LOC), :] = jnp.dot(
                    xcb, wk, preferred_element_type=F32).astype(BF)
                vf[b, pl.ds(row0, S_LOC), :] = jnp.dot(
                    xcb, wv, preferred_element_type=F32).astype(BF)

        qkv_chunk(xbf_buf[...], my * S_LOC)

        for r in (2, 0, 1):
            recv = pltpu.make_async_remote_copy(
                src_ref=xbf_buf,
                dst_ref=xg.at[r],
                send_sem=ag_send.at[0],
                recv_sem=ag_recv.at[r],
                device_id=(my,),
                device_id_type=pl.DeviceIdType.MESH,
            )
            recv.wait_recv()
            origin = lax.rem(my + r + 1, N_DEV)
            qkv_chunk(xg[r], origin * S_LOC)

        def attn_chunk(row0):
            ys = []
            for b in range(B):
                qc = qf[b, pl.ds(row0, S_LOC), :]
                for h in range(H_LOC):
                    q = qc[:, h * DH:(h + 1) * DH]
                    k_ = kf[b, :, h * DH:(h + 1) * DH]
                    v = vf[b, :, h * DH:(h + 1) * DH]
                    s = lax.dot_general(
                        q, k_, (((1,), (1,)), ((), ())),
                        preferred_element_type=F32,
                    ) * 0.125
                    p = jnp.exp(s)
                    l = jnp.sum(p, axis=1, keepdims=True)
                    o = jnp.dot(p.astype(BF), v, preferred_element_type=F32)
                    osc[b, :, h * DH:(h + 1) * DH] = (o / l).astype(BF)
                ys.append(jnp.dot(osc[b], wo, preferred_element_type=F32))
            return ys

        rs_rdmas = []
        for k in (2, 1, 3):
            r = 3 - k
            tgt = lax.rem(my + k, N_DEV)
            ys = attn_chunk(tgt * S_LOC)
            for b in range(B):
                ysend[k - 1, b] = ys[b].astype(BF)
            rdma = pltpu.make_async_remote_copy(
                src_ref=ysend.at[k - 1],
                dst_ref=rsbuf.at[r],
                send_sem=rs_send.at[k - 1],
                recv_sem=rs_recv.at[r],
                device_id=(tgt,),
                device_id_type=pl.DeviceIdType.MESH,
            )
            rdma.start()
            rs_rdmas.append(rdma)

        ys_own = attn_chunk(my * S_LOC)
        for r in (2, 0, 1):
            recv = pltpu.make_async_remote_copy(
                src_ref=ysend.at[0],
                dst_ref=rsbuf.at[r],
                send_sem=rs_send.at[0],
                recv_sem=rs_recv.at[r],
                device_id=(my,),
                device_id_type=pl.DeviceIdType.MESH,
            )
            recv.wait_recv()
        for b in range(B):
            out_ref[b] = ys_own[b] + (
                rsbuf[0, b].astype(F32)
                + rsbuf[1, b].astype(F32)
                + rsbuf[2, b].astype(F32)
            )

        for rdma in ag_rdmas + rs_rdmas:
            rdma.wait_send()

    return pl.pallas_call(
        body,
        out_shape=jax.ShapeDtypeStruct((B, S_LOC, D), F32),
        in_specs=[pl.BlockSpec(memory_space=pltpu.VMEM)] * 5,
        out_specs=pl.BlockSpec(memory_space=pltpu.VMEM),
        scratch_shapes=[
            pltpu.VMEM((B, S_LOC, D), BF),
            pltpu.VMEM((3, B, S_LOC, D), BF),
            pltpu.VMEM((B, S, D), BF),
            pltpu.VMEM((B, S, D), BF),
            pltpu.VMEM((B, S, D), BF),
            pltpu.VMEM((B, S_LOC, D), BF),
            pltpu.VMEM((3, B, S_LOC, D), BF),
            pltpu.VMEM((3, B, S_LOC, D), BF),
            pltpu.SemaphoreType.DMA((3,)),
            pltpu.SemaphoreType.DMA((3,)),
            pltpu.SemaphoreType.DMA((3,)),
            pltpu.SemaphoreType.DMA((3,)),
        ],
        compiler_params=pltpu.CompilerParams(collective_id=0),
    )(x, Wq, Wo, Wk, Wv)


# device time: 27689 ns/iter; 1.2674x vs baseline; 1.2674x over previous
import jax
import jax.numpy as jnp
from jax import lax
from jax.experimental import pallas as pl
from jax.experimental.pallas import tpu as pltpu

N_DEV = 4
B = 2
S_LOC = 128
S = S_LOC * N_DEV
D = 512
H_LOC = 8
DH = 64

BF = jnp.bfloat16
F32 = jnp.float32

ROW_OF_SLOT = {2: 1 * S_LOC, 0: 2 * S_LOC, 1: 3 * S_LOC}
ROW_OF_CHUNK = {3: 1 * S_LOC, 1: 2 * S_LOC, 2: 3 * S_LOC}


def kernel(x, Wq, Wo, Wk, Wv):
    xbf = x.astype(BF)
    wqkv_host = jnp.concatenate(
        [Wq * 0.125, Wk, Wv], axis=1).astype(BF)
    wo_host = Wo.astype(BF)

    def body(x_ref, wqkv_ref, wo_ref, out_ref,
             xbf_buf, xg, qf, kf, vf, osc, y_own, ysend, rsbuf,
             ag_send, ag_recv, rs_send, rs_recv):
        my = lax.axis_index("i")

        barrier_sem = pltpu.get_barrier_semaphore()
        for k in (1, 2, 3):
            pl.semaphore_signal(
                barrier_sem, inc=1,
                device_id=(lax.rem(my + k, N_DEV),),
                device_id_type=pl.DeviceIdType.MESH,
            )
        pl.semaphore_wait(barrier_sem, 3)

        xbf_buf[...] = x_ref[...]
        rdmas = []
        for b in range(B):
            for k in (1, 2, 3):
                r = 3 - k
                rdma = pltpu.make_async_remote_copy(
                    src_ref=xbf_buf.at[b],
                    dst_ref=xg.at[r, b],
                    send_sem=ag_send.at[k - 1, b],
                    recv_sem=ag_recv.at[r, b],
                    device_id=(lax.rem(my + k, N_DEV),),
                    device_id_type=pl.DeviceIdType.MESH,
                )
                rdma.start()
                rdmas.append(rdma)

        wqkv = wqkv_ref[...]
        wo = wo_ref[...]
        ones_l = jnp.ones((S, DH), BF)

        def qkv_chunk(x_cb, b, row0):
            r = jnp.dot(x_cb, wqkv, preferred_element_type=F32).astype(BF)
            qf[b, row0:row0 + S_LOC, :] = r[:, :D]
            kf[b, row0:row0 + S_LOC, :] = r[:, D:2 * D]
            vf[b, row0:row0 + S_LOC, :] = r[:, 2 * D:]

        def wait_ag(r, b):
            recv = pltpu.make_async_remote_copy(
                src_ref=xbf_buf.at[b],
                dst_ref=xg.at[r, b],
                send_sem=ag_send.at[0, b],
                recv_sem=ag_recv.at[r, b],
                device_id=(my,),
                device_id_type=pl.DeviceIdType.MESH,
            )
            recv.wait_recv()

        for b in range(B):
            qkv_chunk(xbf_buf[b], b, 0)

        def attn_batch(b):
            qb = qf[b]
            kb = kf[b]
            vb = vf[b]
            for h in range(H_LOC):
                q = qb[:, h * DH:(h + 1) * DH]
                k_ = kb[:, h * DH:(h + 1) * DH]
                v = vb[:, h * DH:(h + 1) * DH]
                s = lax.dot_general(
                    q, k_, (((1,), (1,)), ((), ())),
                    preferred_element_type=F32,
                )
                p = jnp.exp(s.astype(BF))
                l64 = jnp.dot(p, ones_l, preferred_element_type=F32)
                o = jnp.dot(p, v, preferred_element_type=F32)
                osc[b, :, h * DH:(h + 1) * DH] = (o / l64).astype(BF)
            yv = jnp.dot(osc[b], wo, preferred_element_type=F32)
            y_own[b] = yv[:S_LOC]
            for k in (1, 2, 3):
                r = 3 - k
                row0 = ROW_OF_CHUNK[k]
                ysend[k - 1, b] = yv[row0:row0 + S_LOC].astype(BF)
                rdma = pltpu.make_async_remote_copy(
                    src_ref=ysend.at[k - 1, b],
                    dst_ref=rsbuf.at[r, b],
                    send_sem=rs_send.at[k - 1, b],
                    recv_sem=rs_recv.at[r, b],
                    device_id=(lax.rem(my + k, N_DEV),),
                    device_id_type=pl.DeviceIdType.MESH,
                )
                rdma.start()
                rdmas.append(rdma)

        for r in (2, 0, 1):
            wait_ag(r, 0)
            qkv_chunk(xg[r, 0], 0, ROW_OF_SLOT[r])
        attn_batch(0)

        for r in (2, 0, 1):
            wait_ag(r, 1)
            qkv_chunk(xg[r, 1], 1, ROW_OF_SLOT[r])
        attn_batch(1)

        for b in range(B):
            for r in (2, 0, 1):
                recv = pltpu.make_async_remote_copy(
                    src_ref=ysend.at[0, b],
                    dst_ref=rsbuf.at[r, b],
                    send_sem=rs_send.at[0, b],
                    recv_sem=rs_recv.at[r, b],
                    device_id=(my,),
                    device_id_type=pl.DeviceIdType.MESH,
                )
                recv.wait_recv()
            out_ref[b] = (
                y_own[b]
                + rsbuf[0, b].astype(F32)
                + rsbuf[1, b].astype(F32)
                + rsbuf[2, b].astype(F32)
            ).astype(BF)

        for rdma in rdmas:
            rdma.wait_send()

    return pl.pallas_call(
        body,
        out_shape=jax.ShapeDtypeStruct((B, S_LOC, D), BF),
        in_specs=[pl.BlockSpec(memory_space=pltpu.VMEM)] * 3,
        out_specs=pl.BlockSpec(memory_space=pltpu.VMEM),
        scratch_shapes=[
            pltpu.VMEM((B, S_LOC, D), BF),
            pltpu.VMEM((3, B, S_LOC, D), BF),
            pltpu.VMEM((B, S, D), BF),
            pltpu.VMEM((B, S, D), BF),
            pltpu.VMEM((B, S, D), BF),
            pltpu.VMEM((B, S, D), BF),
            pltpu.VMEM((B, S_LOC, D), F32),
            pltpu.VMEM((3, B, S_LOC, D), BF),
            pltpu.VMEM((3, B, S_LOC, D), BF),
            pltpu.SemaphoreType.DMA((3, B)),
            pltpu.SemaphoreType.DMA((3, B)),
            pltpu.SemaphoreType.DMA((3, B)),
            pltpu.SemaphoreType.DMA((3, B)),
        ],
        compiler_params=pltpu.CompilerParams(collective_id=0),
    )(xbf, wqkv_host, wo_host)
